# baseline (speedup 1.0000x reference)
import functools

import jax
import jax.numpy as jnp
from jax import lax
from jax.experimental import pallas as pl
from jax.experimental.pallas import tpu as pltpu
from jax.experimental.pallas import tpu_sc as plsc


def _build_fm_kernel(B, n_rows, Dp, D):
  info = plsc.get_sparse_core_info()
  NC, NS, L = info.num_cores, info.num_subcores, info.num_lanes
  NW = NC * NS
  bpw = B // NW
  CH = 128
  nch = bpw // CH
  nblk = bpw // L

  mesh = plsc.VectorSubcoreMesh(core_axis_name="c", subcore_axis_name="s")

  @functools.partial(
      pl.kernel,
      mesh=mesh,
      out_type=jax.ShapeDtypeStruct((B,), jnp.float32),
      compiler_params=pltpu.CompilerParams(
          needs_layout_passes=False, use_tc_tiling_on_sc=False),
      scratch_types=[
          pltpu.VMEM((nch, CH), jnp.int32),
          pltpu.VMEM((nch, CH), jnp.int32),
          pltpu.VMEM((bpw, Dp), jnp.float32),
          pltpu.VMEM((bpw, Dp), jnp.float32),
          pltpu.VMEM((bpw,), jnp.float32),
          pltpu.VMEM((L,), jnp.float32),
          pltpu.SemaphoreType.DMA,
      ],
  )
  def fm(i_hbm, j_hbm, t_hbm, c_hbm, out_hbm,
         ii_v, jj_v, vi_v, vj_v, o_v, c_v, sem):
    wid = lax.axis_index("s") * NC + lax.axis_index("c")
    base = wid * bpw

    pltpu.sync_copy(c_hbm, c_v)
    for k in range(nch):
      pltpu.sync_copy(i_hbm.at[pl.ds(base + k * CH, CH)], ii_v.at[k])
      pltpu.sync_copy(j_hbm.at[pl.ds(base + k * CH, CH)], jj_v.at[k])

    copies = []
    for k in range(nch):
      sl = pl.ds(k * CH, CH)
      copies.append(pltpu.async_copy(t_hbm.at[ii_v.at[k]], vi_v.at[sl], sem))
      copies.append(pltpu.async_copy(t_hbm.at[jj_v.at[k]], vj_v.at[sl], sem))
    for cp in copies:
      cp.wait()

    iota = lax.iota(jnp.int32, L)
    cv = c_v[...]
    bcol = jnp.full((L,), D, dtype=jnp.int32)

    def blk_body(b, carry):
      rows = iota + b * L

      def d_body(d, acc):
        cols = jnp.full((L,), d, dtype=jnp.int32)
        a = plsc.load_gather(vi_v, [rows, cols])
        bb = plsc.load_gather(vj_v, [rows, cols])
        return acc + a * bb

      acc = lax.fori_loop(0, D, d_body, jnp.zeros((L,), jnp.float32),
                          unroll=4)
      bi = plsc.load_gather(vi_v, [rows, bcol])
      bj = plsc.load_gather(vj_v, [rows, bcol])
      o_v[pl.ds(b * L, L)] = cv + bi + bj + acc
      return carry

    lax.fori_loop(0, nblk, blk_body, 0)
    pltpu.sync_copy(o_v, out_hbm.at[pl.ds(base, bpw)])

  return fm


def kernel(i, j, y, V, bias, c):
  del y
  B = i.shape[0]
  n_rows, D = V.shape
  Dp = ((D + 1 + 7) // 8) * 8
  table = jnp.concatenate(
      [V, bias.astype(jnp.float32),
       jnp.zeros((n_rows, Dp - D - 1), jnp.float32)], axis=1)
  fm = _build_fm_kernel(B, n_rows, Dp, D)
  c16 = jnp.broadcast_to(c.astype(jnp.float32), (16,))
  return fm(i.astype(jnp.int32), j.astype(jnp.int32), table, c16)


# speedup vs baseline: 2.1648x; 2.1648x over previous
import functools

import jax
import jax.numpy as jnp
from jax import lax
from jax.experimental import pallas as pl
from jax.experimental.pallas import tpu as pltpu
from jax.experimental.pallas import tpu_sc as plsc


def _build_pack_kernel(n_rows, D, Dp, BI):
  grid = (n_rows + BI - 1) // BI

  def pack_body(vt_ref, b_ref, out_ref):
    x = vt_ref[...]
    xt = jnp.transpose(x, (1, 0))
    b = b_ref[...].reshape(BI, 1)
    out_ref[...] = jnp.concatenate(
        [xt[:, :D], b, jnp.zeros((BI, Dp - D - 1), jnp.float32)], axis=1)

  return pl.pallas_call(
      pack_body,
      grid=(grid,),
      in_specs=[
          pl.BlockSpec((Dp, BI), lambda g: (0, g)),
          pl.BlockSpec((BI,), lambda g: (g,)),
      ],
      out_specs=pl.BlockSpec((BI, Dp), lambda g: (g, 0)),
      out_shape=jax.ShapeDtypeStruct((n_rows, Dp), jnp.float32),
      compiler_params=pltpu.CompilerParams(
          dimension_semantics=("arbitrary",)),
  )


def _build_fm_kernel(B, n_rows, Dp, D):
  info = plsc.get_sparse_core_info()
  NC, NS, L = info.num_cores, info.num_subcores, info.num_lanes
  NW = NC * NS
  bpw = B // NW
  CH = 128
  nch = bpw // CH
  nblk = bpw // L

  mesh = plsc.VectorSubcoreMesh(core_axis_name="c", subcore_axis_name="s")

  @functools.partial(
      pl.kernel,
      mesh=mesh,
      out_type=jax.ShapeDtypeStruct((B,), jnp.float32),
      compiler_params=pltpu.CompilerParams(
          needs_layout_passes=False, use_tc_tiling_on_sc=False),
      scratch_types=[
          pltpu.VMEM((nch, CH), jnp.int32),
          pltpu.VMEM((nch, CH), jnp.int32),
          pltpu.VMEM((bpw, Dp), jnp.float32),
          pltpu.VMEM((bpw, Dp), jnp.float32),
          pltpu.VMEM((bpw,), jnp.float32),
          pltpu.VMEM((L,), jnp.float32),
          pltpu.SemaphoreType.DMA,
      ],
  )
  def fm(i_hbm, j_hbm, t_hbm, c_hbm, out_hbm,
         ii_v, jj_v, vi_v, vj_v, o_v, c_v, sem):
    wid = lax.axis_index("s") * NC + lax.axis_index("c")
    base = wid * bpw

    pltpu.sync_copy(c_hbm, c_v)
    for k in range(nch):
      pltpu.sync_copy(i_hbm.at[pl.ds(base + k * CH, CH)], ii_v.at[k])
      pltpu.sync_copy(j_hbm.at[pl.ds(base + k * CH, CH)], jj_v.at[k])

    copies = []
    for k in range(nch):
      sl = pl.ds(k * CH, CH)
      copies.append(pltpu.async_copy(t_hbm.at[ii_v.at[k]], vi_v.at[sl], sem))
      copies.append(pltpu.async_copy(t_hbm.at[jj_v.at[k]], vj_v.at[sl], sem))
    for cp in copies:
      cp.wait()

    iota = lax.iota(jnp.int32, L)
    cv = c_v[...]
    bcol = jnp.full((L,), D, dtype=jnp.int32)

    def blk_body(b, carry):
      rows = iota + b * L

      def d_body(d, acc):
        cols = jnp.full((L,), d, dtype=jnp.int32)
        a = plsc.load_gather(vi_v, [rows, cols])
        bb = plsc.load_gather(vj_v, [rows, cols])
        return acc + a * bb

      acc = lax.fori_loop(0, D, d_body, jnp.zeros((L,), jnp.float32),
                          unroll=4)
      bi = plsc.load_gather(vi_v, [rows, bcol])
      bj = plsc.load_gather(vj_v, [rows, bcol])
      o_v[pl.ds(b * L, L)] = cv + bi + bj + acc
      return carry

    lax.fori_loop(0, nblk, blk_body, 0)
    pltpu.sync_copy(o_v, out_hbm.at[pl.ds(base, bpw)])

  return fm


def kernel(i, j, y, V, bias, c):
  del y
  B = i.shape[0]
  n_rows, D = V.shape
  Dp = ((D + 1 + 7) // 8) * 8
  BI = 2048
  pack = _build_pack_kernel(n_rows, D, Dp, BI)
  table = pack(jnp.transpose(V), bias.reshape(n_rows))
  fm = _build_fm_kernel(B, n_rows, Dp, D)
  c16 = jnp.broadcast_to(c.astype(jnp.float32), (16,))
  return fm(i.astype(jnp.int32), j.astype(jnp.int32), table, c16)


# speedup vs baseline: 2.1662x; 1.0007x over previous
import functools

import jax
import jax.numpy as jnp
from jax import lax
from jax.experimental import pallas as pl
from jax.experimental.pallas import tpu as pltpu
from jax.experimental.pallas import tpu_sc as plsc


def _build_pack_kernel(n_rows, D, Dp, BI):
  grid = (n_rows + BI - 1) // BI

  def pack_body(vt_ref, b_ref, out_ref):
    x = vt_ref[...]
    xt = jnp.transpose(x, (1, 0))
    out_ref[:, :D] = xt[:, :D]
    out_ref[:, D:D + 1] = b_ref[...].reshape(BI, 1)

  return pl.pallas_call(
      pack_body,
      grid=(grid,),
      in_specs=[
          pl.BlockSpec((Dp, BI), lambda g: (0, g)),
          pl.BlockSpec((BI,), lambda g: (g,)),
      ],
      out_specs=pl.BlockSpec((BI, Dp), lambda g: (g, 0)),
      out_shape=jax.ShapeDtypeStruct((n_rows, Dp), jnp.float32),
      compiler_params=pltpu.CompilerParams(
          dimension_semantics=("parallel",)),
  )


def _build_fm_kernel(B, n_rows, Dp, D):
  info = plsc.get_sparse_core_info()
  NC, NS, L = info.num_cores, info.num_subcores, info.num_lanes
  NW = NC * NS
  bpw = B // NW
  CH = 128
  nch = bpw // CH
  nblk = bpw // L

  mesh = plsc.VectorSubcoreMesh(core_axis_name="c", subcore_axis_name="s")

  @functools.partial(
      pl.kernel,
      mesh=mesh,
      out_type=jax.ShapeDtypeStruct((B,), jnp.float32),
      compiler_params=pltpu.CompilerParams(
          needs_layout_passes=False, use_tc_tiling_on_sc=False),
      scratch_types=[
          pltpu.VMEM((nch, CH), jnp.int32),
          pltpu.VMEM((nch, CH), jnp.int32),
          pltpu.VMEM((bpw, Dp), jnp.float32),
          pltpu.VMEM((bpw, Dp), jnp.float32),
          pltpu.VMEM((bpw,), jnp.float32),
          pltpu.VMEM((L,), jnp.float32),
          pltpu.SemaphoreType.DMA,
      ],
  )
  def fm(i_hbm, j_hbm, t_hbm, c_hbm, out_hbm,
         ii_v, jj_v, vi_v, vj_v, o_v, c_v, sem):
    wid = lax.axis_index("s") * NC + lax.axis_index("c")
    base = wid * bpw

    pltpu.sync_copy(c_hbm, c_v)
    for k in range(nch):
      pltpu.sync_copy(i_hbm.at[pl.ds(base + k * CH, CH)], ii_v.at[k])
      pltpu.sync_copy(j_hbm.at[pl.ds(base + k * CH, CH)], jj_v.at[k])

    copies = []
    for k in range(nch):
      sl = pl.ds(k * CH, CH)
      copies.append(pltpu.async_copy(t_hbm.at[ii_v.at[k]], vi_v.at[sl], sem))
      copies.append(pltpu.async_copy(t_hbm.at[jj_v.at[k]], vj_v.at[sl], sem))
    for cp in copies:
      cp.wait()

    iota = lax.iota(jnp.int32, L)
    cv = c_v[...]
    bcol = jnp.full((L,), D, dtype=jnp.int32)

    def blk_body(b, carry):
      rows = iota + b * L

      def d_body(d, acc):
        cols = jnp.full((L,), d, dtype=jnp.int32)
        a = plsc.load_gather(vi_v, [rows, cols])
        bb = plsc.load_gather(vj_v, [rows, cols])
        return acc + a * bb

      acc = lax.fori_loop(0, D, d_body, jnp.zeros((L,), jnp.float32),
                          unroll=4)
      bi = plsc.load_gather(vi_v, [rows, bcol])
      bj = plsc.load_gather(vj_v, [rows, bcol])
      o_v[pl.ds(b * L, L)] = cv + bi + bj + acc
      return carry

    lax.fori_loop(0, nblk, blk_body, 0)
    pltpu.sync_copy(o_v, out_hbm.at[pl.ds(base, bpw)])

  return fm


def kernel(i, j, y, V, bias, c):
  del y
  B = i.shape[0]
  n_rows, D = V.shape
  Dp = ((D + 1 + 7) // 8) * 8
  BI = 2048
  pack = _build_pack_kernel(n_rows, D, Dp, BI)
  table = pack(jnp.transpose(V), bias.reshape(n_rows))
  fm = _build_fm_kernel(B, n_rows, Dp, D)
  c16 = jnp.broadcast_to(c.astype(jnp.float32), (16,))
  return fm(i.astype(jnp.int32), j.astype(jnp.int32), table, c16)


# speedup vs baseline: 4.0178x; 1.8548x over previous
import functools

import jax
import jax.numpy as jnp
from jax import lax
from jax.experimental import pallas as pl
from jax.experimental.pallas import tpu as pltpu
from jax.experimental.pallas import tpu_sc as plsc


def _build_pack_kernel(n_rows, D, Dp, BI):
  grid = (n_rows + BI - 1) // BI
  Din = ((D + 7) // 8) * 8

  def pack_body(vt_ref, b_ref, out_ref):
    x = vt_ref[...]
    xt = jnp.transpose(x, (1, 0))
    out_ref[:, :D] = xt[:, :D]
    out_ref[:, D:D + 1] = b_ref[...].reshape(BI, 1)

  return pl.pallas_call(
      pack_body,
      grid=(grid,),
      in_specs=[
          pl.BlockSpec((Din, BI), lambda g: (0, g)),
          pl.BlockSpec((BI,), lambda g: (g,)),
      ],
      out_specs=pl.BlockSpec((BI, Dp), lambda g: (g, 0)),
      out_shape=jax.ShapeDtypeStruct((n_rows, Dp), jnp.float32),
      compiler_params=pltpu.CompilerParams(
          dimension_semantics=("parallel",)),
  )


def _build_fm_kernel(B, n_rows, Dp, D):
  info = plsc.get_sparse_core_info()
  NC, NS, L = info.num_cores, info.num_subcores, info.num_lanes
  NW = NC * NS
  bpw = B // NW
  CH = 128
  nch = bpw // CH
  nblk = CH // L

  mesh = plsc.VectorSubcoreMesh(core_axis_name="c", subcore_axis_name="s")

  @functools.partial(
      pl.kernel,
      mesh=mesh,
      out_type=jax.ShapeDtypeStruct((B,), jnp.float32),
      compiler_params=pltpu.CompilerParams(
          needs_layout_passes=False, use_tc_tiling_on_sc=False),
      scratch_types=[
          pltpu.VMEM((nch, CH), jnp.int32),
          pltpu.VMEM((nch, CH), jnp.int32),
          pltpu.VMEM((2, CH, Dp), jnp.float32),
          pltpu.VMEM((2, CH, Dp), jnp.float32),
          pltpu.VMEM((bpw,), jnp.float32),
          pltpu.VMEM((L,), jnp.float32),
          pltpu.SemaphoreType.DMA,
          pltpu.SemaphoreType.DMA,
      ],
  )
  def fm(i_hbm, j_hbm, t_hbm, c_hbm, out_hbm,
         ii_v, jj_v, vi_v, vj_v, o_v, c_v, sem0, sem1):
    wid = lax.axis_index("s") * NC + lax.axis_index("c")
    base = wid * bpw
    sems = (sem0, sem1)

    pltpu.sync_copy(c_hbm, c_v)
    for k in range(nch):
      pltpu.sync_copy(i_hbm.at[pl.ds(base + k * CH, CH)], ii_v.at[k])
      pltpu.sync_copy(j_hbm.at[pl.ds(base + k * CH, CH)], jj_v.at[k])

    def start(k):
      p = k % 2
      return (
          pltpu.async_copy(t_hbm.at[ii_v.at[k]], vi_v.at[p], sems[p]),
          pltpu.async_copy(t_hbm.at[jj_v.at[k]], vj_v.at[p], sems[p]),
      )

    iota = lax.iota(jnp.int32, L)
    cv = c_v[...]
    bcol = jnp.full((L,), D, dtype=jnp.int32)

    inflight = start(0)
    for k in range(nch):
      for cp in inflight:
        cp.wait()
      if k + 1 < nch:
        inflight = start(k + 1)
      p = k % 2
      pvec = jnp.full((L,), p, dtype=jnp.int32)

      def blk_body(b, carry):
        rows = iota + b * L

        def d_body(d, acc):
          cols = jnp.full((L,), d, dtype=jnp.int32)
          a = plsc.load_gather(vi_v, [pvec, rows, cols])
          bb = plsc.load_gather(vj_v, [pvec, rows, cols])
          return acc + a * bb

        acc = lax.fori_loop(0, D, d_body, jnp.zeros((L,), jnp.float32),
                            unroll=4)
        bi = plsc.load_gather(vi_v, [pvec, rows, bcol])
        bj = plsc.load_gather(vj_v, [pvec, rows, bcol])
        o_v[pl.ds(k * CH + b * L, L)] = cv + bi + bj + acc
        return carry

      lax.fori_loop(0, nblk, blk_body, 0)

    pltpu.sync_copy(o_v, out_hbm.at[pl.ds(base, bpw)])

  return fm


def kernel(i, j, y, V, bias, c):
  del y
  B = i.shape[0]
  n_rows, D = V.shape
  Dp = 128
  BI = 2048
  pack = _build_pack_kernel(n_rows, D, Dp, BI)
  table = pack(jnp.transpose(V), bias.reshape(n_rows))
  fm = _build_fm_kernel(B, n_rows, Dp, D)
  c16 = jnp.broadcast_to(c.astype(jnp.float32), (16,))
  return fm(i.astype(jnp.int32), j.astype(jnp.int32), table, c16)


# speedup vs baseline: 5.7380x; 1.4281x over previous
import functools

import jax
import jax.numpy as jnp
from jax import lax
from jax.experimental import pallas as pl
from jax.experimental.pallas import tpu as pltpu
from jax.experimental.pallas import tpu_sc as plsc


def _build_pack_kernel(n_rows, D, Dp, BI):
  grid = (n_rows + BI - 1) // BI
  Din = ((D + 7) // 8) * 8

  def pack_body(vt_ref, b_ref, out_ref):
    x = vt_ref[...]
    xt = jnp.transpose(x, (1, 0))
    out_ref[:, :D] = xt[:, :D]
    out_ref[:, D:D + 1] = b_ref[...].reshape(BI, 1)

  return pl.pallas_call(
      pack_body,
      grid=(grid,),
      in_specs=[
          pl.BlockSpec((Din, BI), lambda g: (0, g)),
          pl.BlockSpec((BI,), lambda g: (g,)),
      ],
      out_specs=pl.BlockSpec((BI, Dp), lambda g: (g, 0)),
      out_shape=jax.ShapeDtypeStruct((n_rows, Dp), jnp.float32),
      compiler_params=pltpu.CompilerParams(
          dimension_semantics=("parallel",)),
  )


def _build_fm_kernel(B, n_rows, Dp, D):
  info = plsc.get_sparse_core_info()
  NC, NS, L = info.num_cores, info.num_subcores, info.num_lanes
  NW = NC * NS
  bpw = B // NW
  CH = 128
  nch = bpw // CH
  nblk = CH // L

  mesh = plsc.VectorSubcoreMesh(core_axis_name="c", subcore_axis_name="s")

  @functools.partial(
      pl.kernel,
      mesh=mesh,
      out_type=jax.ShapeDtypeStruct((B,), jnp.float32),
      compiler_params=pltpu.CompilerParams(
          needs_layout_passes=False, use_tc_tiling_on_sc=False),
      scratch_types=[
          pltpu.VMEM((nch, CH), jnp.int32),
          pltpu.VMEM((nch, CH), jnp.int32),
          pltpu.VMEM((2, CH, Dp), jnp.float32),
          pltpu.VMEM((2, CH, Dp), jnp.float32),
          pltpu.VMEM((bpw,), jnp.float32),
          pltpu.VMEM((L,), jnp.float32),
          pltpu.SemaphoreType.DMA,
          pltpu.SemaphoreType.DMA,
      ],
  )
  def fm(i_hbm, j_hbm, t_hbm, c_hbm, out_hbm,
         ii_v, jj_v, vi_v, vj_v, o_v, c_v, sem0, sem1):
    wid = lax.axis_index("s") * NC + lax.axis_index("c")
    base = wid * bpw
    sems = (sem0, sem1)

    pltpu.sync_copy(c_hbm, c_v)
    for k in range(nch):
      pltpu.sync_copy(i_hbm.at[pl.ds(base + k * CH, CH)], ii_v.at[k])
      pltpu.sync_copy(j_hbm.at[pl.ds(base + k * CH, CH)], jj_v.at[k])

    def start(k):
      p = k % 2
      return (
          pltpu.async_copy(t_hbm.at[ii_v.at[k]], vi_v.at[p], sems[p]),
          pltpu.async_copy(t_hbm.at[jj_v.at[k]], vj_v.at[p], sems[p]),
      )

    iota = lax.iota(jnp.int32, L)
    cv = c_v[...]
    bcol = jnp.full((L,), D, dtype=jnp.int32)

    inflight = start(0)
    for k in range(nch):
      for cp in inflight:
        cp.wait()
      if k + 1 < nch:
        inflight = start(k + 1)
      p = k % 2
      pvec = jnp.full((L,), p, dtype=jnp.int32)

      def blk_body(b, carry):
        rows = iota + b * L

        def d_body(d, acc):
          cols = jnp.full((L,), d, dtype=jnp.int32)
          a = plsc.load_gather(vi_v, [pvec, rows, cols])
          bb = plsc.load_gather(vj_v, [pvec, rows, cols])
          return acc + a * bb

        acc = lax.fori_loop(0, D, d_body, jnp.zeros((L,), jnp.float32),
                            unroll=4)
        bi = plsc.load_gather(vi_v, [pvec, rows, bcol])
        bj = plsc.load_gather(vj_v, [pvec, rows, bcol])
        o_v[pl.ds(k * CH + b * L, L)] = cv + bi + bj + acc
        return carry

      lax.fori_loop(0, nblk, blk_body, 0)

    pltpu.sync_copy(o_v, out_hbm.at[pl.ds(base, bpw)])

  return fm


def kernel(i, j, y, V, bias, c):
  del y
  B = i.shape[0]
  n_rows, D = V.shape
  Dp = 128
  BI = 8192
  pack = _build_pack_kernel(n_rows, D, Dp, BI)
  table = pack(jnp.transpose(V), bias.reshape(n_rows))
  fm = _build_fm_kernel(B, n_rows, Dp, D)
  c16 = jnp.broadcast_to(c.astype(jnp.float32), (16,))
  return fm(i.astype(jnp.int32), j.astype(jnp.int32), table, c16)


# speedup vs baseline: 6.1953x; 1.0797x over previous
import functools

import jax
import jax.numpy as jnp
from jax import lax
from jax.experimental import pallas as pl
from jax.experimental.pallas import tpu as pltpu
from jax.experimental.pallas import tpu_sc as plsc


def _build_pack_kernel(n_rows, D, Dp, BI):
  grid = (n_rows + BI - 1) // BI
  Din = ((D + 7) // 8) * 8

  def pack_body(vt_ref, b_ref, out_ref):
    x = vt_ref[...]
    xt = jnp.transpose(x, (1, 0))
    out_ref[:, :D] = xt[:, :D]
    out_ref[:, D:D + 1] = b_ref[...].reshape(BI, 1)

  return pl.pallas_call(
      pack_body,
      grid=(grid,),
      in_specs=[
          pl.BlockSpec((Din, BI), lambda g: (0, g)),
          pl.BlockSpec((BI,), lambda g: (g,)),
      ],
      out_specs=pl.BlockSpec((BI, Dp), lambda g: (g, 0)),
      out_shape=jax.ShapeDtypeStruct((n_rows, Dp), jnp.float32),
      compiler_params=pltpu.CompilerParams(
          dimension_semantics=("parallel",)),
  )


def _build_fm_kernel(B, n_rows, Dp, D):
  info = plsc.get_sparse_core_info()
  NC, NS, L = info.num_cores, info.num_subcores, info.num_lanes
  NW = NC * NS
  bpw = B // NW
  CH = 128
  nch = bpw // CH
  nblk = CH // L

  mesh = plsc.VectorSubcoreMesh(core_axis_name="c", subcore_axis_name="s")

  @functools.partial(
      pl.kernel,
      mesh=mesh,
      out_type=jax.ShapeDtypeStruct((B,), jnp.float32),
      compiler_params=pltpu.CompilerParams(
          needs_layout_passes=False, use_tc_tiling_on_sc=False),
      scratch_types=[
          pltpu.VMEM((nch, CH), jnp.int32),
          pltpu.VMEM((nch, CH), jnp.int32),
          pltpu.VMEM((2, CH, Dp), jnp.float32),
          pltpu.VMEM((2, CH, Dp), jnp.float32),
          pltpu.VMEM((bpw,), jnp.float32),
          pltpu.VMEM((L,), jnp.float32),
          pltpu.SemaphoreType.DMA,
          pltpu.SemaphoreType.DMA,
      ],
  )
  def fm(i_hbm, j_hbm, t_hbm, c_hbm, out_hbm,
         ii_v, jj_v, vi_v, vj_v, o_v, c_v, sem0, sem1):
    wid = lax.axis_index("s") * NC + lax.axis_index("c")
    base = wid * bpw
    sems = (sem0, sem1)

    pltpu.sync_copy(c_hbm, c_v)
    for k in range(nch):
      pltpu.sync_copy(i_hbm.at[pl.ds(base + k * CH, CH)], ii_v.at[k])
      pltpu.sync_copy(j_hbm.at[pl.ds(base + k * CH, CH)], jj_v.at[k])

    def start(k):
      p = k % 2
      return (
          pltpu.async_copy(t_hbm.at[ii_v.at[k]], vi_v.at[p], sems[p]),
          pltpu.async_copy(t_hbm.at[jj_v.at[k]], vj_v.at[p], sems[p]),
      )

    iota = lax.iota(jnp.int32, L)
    cv = c_v[...]
    bcol = jnp.full((L,), D, dtype=jnp.int32)

    inflight = start(0)
    for k in range(nch):
      for cp in inflight:
        cp.wait()
      if k + 1 < nch:
        inflight = start(k + 1)
      p = k % 2
      pvec = jnp.full((L,), p, dtype=jnp.int32)

      def blk_body(b, carry):
        rows = iota + b * L

        def d_body(d, acc):
          cols = jnp.full((L,), d, dtype=jnp.int32)
          a = plsc.load_gather(vi_v, [pvec, rows, cols])
          bb = plsc.load_gather(vj_v, [pvec, rows, cols])
          return acc + a * bb

        acc = lax.fori_loop(0, D, d_body, jnp.zeros((L,), jnp.float32),
                            unroll=4)
        bi = plsc.load_gather(vi_v, [pvec, rows, bcol])
        bj = plsc.load_gather(vj_v, [pvec, rows, bcol])
        o_v[pl.ds(k * CH + b * L, L)] = cv + bi + bj + acc
        return carry

      lax.fori_loop(0, nblk, blk_body, 0)

    pltpu.sync_copy(o_v, out_hbm.at[pl.ds(base, bpw)])

  return fm


def kernel(i, j, y, V, bias, c):
  del y
  B = i.shape[0]
  n_rows, D = V.shape
  Dp = 128
  BI = 16384
  pack = _build_pack_kernel(n_rows, D, Dp, BI)
  table = pack(jnp.transpose(V), bias.reshape(n_rows))
  fm = _build_fm_kernel(B, n_rows, Dp, D)
  c16 = jnp.broadcast_to(c.astype(jnp.float32), (16,))
  return fm(i.astype(jnp.int32), j.astype(jnp.int32), table, c16)


# speedup vs baseline: 6.4007x; 1.0332x over previous
import functools

import jax
import jax.numpy as jnp
from jax import lax
from jax.experimental import pallas as pl
from jax.experimental.pallas import tpu as pltpu
from jax.experimental.pallas import tpu_sc as plsc


def _build_pack_kernel(n_rows, D, Dp, BI):
  grid = (n_rows + BI - 1) // BI
  Din = ((D + 7) // 8) * 8

  def pack_body(vt_ref, b_ref, out_ref):
    x = vt_ref[...]
    xt = jnp.transpose(x, (1, 0))
    out_ref[:, :D] = xt[:, :D]
    out_ref[:, D:D + 1] = b_ref[...].reshape(BI, 1)

  return pl.pallas_call(
      pack_body,
      grid=(grid,),
      in_specs=[
          pl.BlockSpec((Din, BI), lambda g: (0, g)),
          pl.BlockSpec((BI,), lambda g: (g,)),
      ],
      out_specs=pl.BlockSpec((BI, Dp), lambda g: (g, 0)),
      out_shape=jax.ShapeDtypeStruct((n_rows, Dp), jnp.float32),
      compiler_params=pltpu.CompilerParams(
          dimension_semantics=("parallel",)),
  )


def _build_fm_kernel(B, n_rows, Dp, D):
  info = plsc.get_sparse_core_info()
  NC, NS, L = info.num_cores, info.num_subcores, info.num_lanes
  NW = NC * NS
  bpw = B // NW
  CH = 128
  nch = bpw // CH
  nblk = CH // L

  mesh = plsc.VectorSubcoreMesh(core_axis_name="c", subcore_axis_name="s")

  @functools.partial(
      pl.kernel,
      mesh=mesh,
      out_type=jax.ShapeDtypeStruct((B,), jnp.float32),
      compiler_params=pltpu.CompilerParams(
          needs_layout_passes=False, use_tc_tiling_on_sc=False),
      scratch_types=[
          pltpu.VMEM((nch, CH), jnp.int32),
          pltpu.VMEM((nch, CH), jnp.int32),
          pltpu.VMEM((2, CH, Dp), jnp.float32),
          pltpu.VMEM((2, CH, Dp), jnp.float32),
          pltpu.VMEM((bpw,), jnp.float32),
          pltpu.VMEM((L,), jnp.float32),
          pltpu.SemaphoreType.DMA,
          pltpu.SemaphoreType.DMA,
      ],
  )
  def fm(i_hbm, j_hbm, t_hbm, c_hbm, out_hbm,
         ii_v, jj_v, vi_v, vj_v, o_v, c_v, sem0, sem1):
    wid = lax.axis_index("s") * NC + lax.axis_index("c")
    base = wid * bpw
    sems = (sem0, sem1)

    pltpu.sync_copy(c_hbm, c_v)
    for k in range(nch):
      pltpu.sync_copy(i_hbm.at[pl.ds(base + k * CH, CH)], ii_v.at[k])
      pltpu.sync_copy(j_hbm.at[pl.ds(base + k * CH, CH)], jj_v.at[k])

    def start(k):
      p = k % 2
      return (
          pltpu.async_copy(t_hbm.at[ii_v.at[k]], vi_v.at[p], sems[p]),
          pltpu.async_copy(t_hbm.at[jj_v.at[k]], vj_v.at[p], sems[p]),
      )

    iota = lax.iota(jnp.int32, L)
    cv = c_v[...]
    bcol = jnp.full((L,), D, dtype=jnp.int32)

    inflight = start(0)
    for k in range(nch):
      for cp in inflight:
        cp.wait()
      if k + 1 < nch:
        inflight = start(k + 1)
      p = k % 2
      pvec = jnp.full((L,), p, dtype=jnp.int32)

      def blk_body(b, carry):
        rows = iota + b * L

        def d_body(d, acc):
          cols = jnp.full((L,), d, dtype=jnp.int32)
          a = plsc.load_gather(vi_v, [pvec, rows, cols])
          bb = plsc.load_gather(vj_v, [pvec, rows, cols])
          return acc + a * bb

        acc = lax.fori_loop(0, D, d_body, jnp.zeros((L,), jnp.float32),
                            unroll=4)
        bi = plsc.load_gather(vi_v, [pvec, rows, bcol])
        bj = plsc.load_gather(vj_v, [pvec, rows, bcol])
        o_v[pl.ds(k * CH + b * L, L)] = cv + bi + bj + acc
        return carry

      lax.fori_loop(0, nblk, blk_body, 0)

    pltpu.sync_copy(o_v, out_hbm.at[pl.ds(base, bpw)])

  return fm


def kernel(i, j, y, V, bias, c):
  del y
  B = i.shape[0]
  n_rows, D = V.shape
  Dp = 128
  BI = 24576
  pack = _build_pack_kernel(n_rows, D, Dp, BI)
  table = pack(jnp.transpose(V), bias.reshape(n_rows))
  fm = _build_fm_kernel(B, n_rows, Dp, D)
  c16 = jnp.broadcast_to(c.astype(jnp.float32), (16,))
  return fm(i.astype(jnp.int32), j.astype(jnp.int32), table, c16)
